# baseline (speedup 1.0000x reference)
import functools

import jax
import jax.numpy as jnp
from jax import lax
from jax.experimental import pallas as pl
from jax.experimental.pallas import tpu as pltpu
from jax.experimental.pallas import tpu_sc as plsc

HIDDEN = 768
BATCH = 4
SEQ = 2048
ROWS = BATCH * SEQ
EPS = 1e-12
_NV = HIDDEN // 16
_CB = 8
_NCB = _NV // _CB

_info = plsc.get_sparse_core_info()
_NC, _NS = _info.num_cores, _info.num_subcores
_NW = _NC * _NS
_RPW = ROWS // _NW
_CH = 16
_NCH = _RPW // _CH
_NTB = 4
_NPB = 2
_GRP = 4

_mesh = plsc.VectorSubcoreMesh(core_axis_name="c", subcore_axis_name="s")


@functools.partial(
    pl.kernel,
    mesh=_mesh,
    out_type=jax.ShapeDtypeStruct((ROWS, HIDDEN), jnp.float32),
    scratch_types=[
        pltpu.VMEM((_RPW,), jnp.int32),
        pltpu.VMEM((HIDDEN,), jnp.float32),
        pltpu.VMEM((HIDDEN,), jnp.float32),
        pltpu.VMEM((_NTB, _CH, HIDDEN), jnp.float32),
        pltpu.VMEM((_NPB, _CH, HIDDEN), jnp.float32),
        pltpu.VMEM((_CH, 16), jnp.float32),
        pltpu.VMEM((_CH, 16), jnp.float32),
        pltpu.VMEM((16,), jnp.float32),
        pltpu.VMEM((16,), jnp.float32),
        pltpu.SemaphoreType.DMA,
        pltpu.SemaphoreType.DMA,
        pltpu.SemaphoreType.DMA,
    ],
)
def _fused_sc(table_hbm, ids_hbm, pos_hbm, gam_hbm, bet_hbm, out_hbm,
              idx_v, gam_v, bet_v, tok_v, pos_v, av_v, bv_v, ts_v, tq_v,
              sem_g, sem_p, sem_o):
    wid = lax.axis_index("s") * _NC + lax.axis_index("c")
    base = wid * _RPW
    pbase = base % SEQ
    pltpu.sync_copy(ids_hbm.at[pl.ds(base, _RPW)], idx_v)
    pltpu.sync_copy(gam_hbm, gam_v)
    pltpu.sync_copy(bet_hbm, bet_v)

    def start_gather(c, slot):
        off = pl.multiple_of(c * _CH, 8)
        pltpu.async_copy(table_hbm.at[idx_v.at[pl.ds(off, _CH)]],
                         tok_v.at[slot], sem_g)

    def start_pos(c, slot):
        pltpu.async_copy(pos_hbm.at[pl.ds(pbase + c * _CH, _CH)],
                         pos_v.at[slot], sem_p)

    def drain_gather():
        pltpu.make_async_copy(pos_hbm.at[pl.ds(0, _CH)], tok_v.at[0], sem_g).wait()

    def drain_pos():
        pltpu.make_async_copy(pos_hbm.at[pl.ds(0, _CH)], pos_v.at[0], sem_p).wait()

    def drain_out():
        pltpu.make_async_copy(tok_v.at[0], out_hbm.at[pl.ds(base, _CH)], sem_o).wait()

    for s in range(_NTB - 1):
        start_gather(s, s)
    for s in range(_NPB):
        start_pos(s, s)

    def group(gi, carry):
        for b in range(_GRP):
            c = gi * _GRP + b
            tokb = tok_v.at[b]
            posb = pos_v.at[b % _NPB]
            drain_gather()
            drain_pos()

            def pass1(r, _):
                acc_s = jnp.zeros((16,), jnp.float32)
                acc_q = jnp.zeros((16,), jnp.float32)
                for j in range(_NV):
                    sl = pl.ds(j * 16, 16)
                    x = tokb[r, sl] + posb[r, sl]
                    tokb[r, sl] = x
                    acc_s = acc_s + x
                    acc_q = acc_q + x * x
                s = acc_s[0]
                q = acc_q[0]
                for l in range(1, 16):
                    s = s + acc_s[l]
                    q = q + acc_q[l]
                mean = s * (1.0 / HIDDEN)
                var = jnp.maximum(q * (1.0 / HIDDEN) - mean * mean, 0.0)
                vv = var + EPS
                ib = lax.bitcast_convert_type(vv, jnp.int32)
                y = lax.bitcast_convert_type(
                    jnp.int32(0x5F3759DF) - lax.shift_right_logical(ib, 1),
                    jnp.float32)
                h = vv * 0.5
                for _n in range(4):
                    y = y * (1.5 - h * y * y)
                av_v[r, :] = jnp.broadcast_to(y, (16,))
                bv_v[r, :] = jnp.broadcast_to(-mean * y, (16,))
                return 0

            lax.fori_loop(0, _CH, pass1, 0, unroll=False)

            @pl.when(c + _NPB < _NCH)
            def _():
                start_pos(c + _NPB, (b + _NPB) % _NPB)

            for cb in range(_NCB):
                gs = [gam_v[pl.ds((cb * _CB + jj) * 16, 16)] for jj in range(_CB)]
                bs = [bet_v[pl.ds((cb * _CB + jj) * 16, 16)] for jj in range(_CB)]

                def pass2(r, _, gs=gs, bs=bs, cb=cb):
                    a = av_v[r, :]
                    bsh = bv_v[r, :]
                    for jj in range(_CB):
                        sl = pl.ds((cb * _CB + jj) * 16, 16)
                        x = tokb[r, sl]
                        tokb[r, sl] = (x * a + bsh) * gs[jj] + bs[jj]
                    return 0

                lax.fori_loop(0, _CH, pass2, 0, unroll=False)

            pltpu.async_copy(tokb, out_hbm.at[pl.ds(base + c * _CH, _CH)], sem_o)

            @pl.when(jnp.logical_and(c >= 1, c + _NTB - 1 < _NCH))
            def _():
                drain_out()
                start_gather(c + _NTB - 1, (b + _NTB - 1) % _NTB)

            @pl.when(jnp.logical_and(c < 1, c + _NTB - 1 < _NCH))
            def _():
                start_gather(c + _NTB - 1, (b + _NTB - 1) % _NTB)
        return carry

    lax.fori_loop(0, _NCH // _GRP, group, 0, unroll=False)
    for _ in range(_NTB):
        drain_out()


def kernel(input_ids, token_table, pos_table, gamma, beta):
    ids = input_ids.reshape(-1).astype(jnp.int32)
    out = _fused_sc(token_table, ids, pos_table, gamma, beta)
    return out.reshape(BATCH, SEQ, HIDDEN)


# speedup vs baseline: 1.5896x; 1.5896x over previous
import functools

import jax
import jax.numpy as jnp
from jax import lax
from jax.experimental import pallas as pl
from jax.experimental.pallas import tpu as pltpu
from jax.experimental.pallas import tpu_sc as plsc

HIDDEN = 768
BATCH = 4
SEQ = 2048
ROWS = BATCH * SEQ
EPS = 1e-12

_info = plsc.get_sparse_core_info()
_NC, _NS = _info.num_cores, _info.num_subcores
_NW = _NC * _NS
_RPW = ROWS // _NW
_CH = 32
_NCH = _RPW // _CH
_NB = 4

_mesh = plsc.VectorSubcoreMesh(core_axis_name="c", subcore_axis_name="s")


@functools.partial(
    pl.kernel,
    mesh=_mesh,
    out_type=jax.ShapeDtypeStruct((ROWS, HIDDEN), jnp.float32),
    scratch_types=[
        pltpu.VMEM((_RPW,), jnp.int32),
        pltpu.VMEM((_NB, _CH, HIDDEN), jnp.float32),
        pltpu.SemaphoreType.DMA,
        pltpu.SemaphoreType.DMA,
    ],
)
def _gather_sc(table_hbm, ids_hbm, out_hbm, idx_v, buf_v, sem_g, sem_o):
    wid = lax.axis_index("s") * _NC + lax.axis_index("c")
    base = wid * _RPW
    pltpu.sync_copy(ids_hbm.at[pl.ds(base, _RPW)], idx_v)

    gathers = [None] * _NCH
    outs = [None] * _NCH

    def start_gather(c):
        gathers[c] = pltpu.async_copy(
            table_hbm.at[idx_v.at[pl.ds(c * _CH, _CH)]],
            buf_v.at[c % _NB], sem_g)

    for c in range(_NB - 1):
        start_gather(c)
    for c in range(_NCH):
        gathers[c].wait()
        outs[c] = pltpu.async_copy(
            buf_v.at[c % _NB], out_hbm.at[pl.ds(base + c * _CH, _CH)], sem_o)
        if c + _NB - 1 < _NCH:
            if c >= 1:
                outs[c - 1].wait()
            start_gather(c + _NB - 1)
    for c in range(_NCH - _NB, _NCH):
        outs[c].wait()


def _ln_body(pos_ref, gamma_ref, beta_ref, emb_ref, out_ref):
    x = emb_ref[...] + pos_ref[...]
    mean = jnp.mean(x, axis=1, keepdims=True)
    xc = x - mean
    var = jnp.mean(xc * xc, axis=1, keepdims=True)
    inv = lax.rsqrt(var + EPS)
    out_ref[...] = xc * inv * gamma_ref[...] + beta_ref[...]


def kernel(input_ids, token_table, pos_table, gamma, beta):
    ids = input_ids.reshape(-1).astype(jnp.int32)
    emb = _gather_sc(token_table, ids)
    r = 2048
    out = pl.pallas_call(
        _ln_body,
        grid=(SEQ // r, BATCH),
        in_specs=[
            pl.BlockSpec((r, HIDDEN), lambda j, b: (j, 0)),
            pl.BlockSpec((1, HIDDEN), lambda j, b: (0, 0)),
            pl.BlockSpec((1, HIDDEN), lambda j, b: (0, 0)),
            pl.BlockSpec((r, HIDDEN), lambda j, b: (b * (SEQ // r) + j, 0)),
        ],
        out_specs=pl.BlockSpec((r, HIDDEN), lambda j, b: (b * (SEQ // r) + j, 0)),
        out_shape=jax.ShapeDtypeStruct((ROWS, HIDDEN), jnp.float32),
    )(pos_table, gamma.reshape(1, HIDDEN), beta.reshape(1, HIDDEN), emb)
    return out.reshape(BATCH, SEQ, HIDDEN)
